# Initial kernel scaffold; baseline (speedup 1.0000x reference)
import jax
import jax.numpy as jnp
from jax.experimental import pallas as pl


def kernel(x, edge_index, p_map, W, b):
    raise NotImplementedError("write your pallas kernel here")


# speedup vs baseline: 13.6382x; 13.6382x over previous
import jax
import jax.numpy as jnp
from jax import lax
from jax.experimental import pallas as pl
from jax.experimental.pallas import tpu as pltpu
from jax.experimental.pallas import tpu_sc as plsc

N = 10000
D = 128
E = 320000
NC = 2
NS = 16
NSPLIT = N // NC
LOCAL_ROWS = 5120
TRASH = NSPLIT
CHUNK = 128
CHUNKS = -(-E // (NS * CHUNK))
E_PAD = NS * CHUNKS * CHUNK
ROWS_PER_TILE = LOCAL_ROWS // NS
SLICE = 352
DEG_ROWS = NS * SLICE
TC_BLK = 1000


def _sc_body(x_hbm, src_hbm, dst_hbm, agg_out, deg_out,
             src_v, dst_v, rows_v, deg_local, dsum_v,
             agg_sh, deg_all, sem_g):
    c = lax.axis_index("c")
    s = lax.axis_index("s")
    base = s * ROWS_PER_TILE

    pltpu.sync_copy(src_hbm.at[s], src_v)
    pltpu.sync_copy(dst_hbm.at[c, s], dst_v)

    zer = jnp.zeros((16,), jnp.float32)
    one = jnp.ones((16,), jnp.float32)

    def fill_rows(r, carry):
        for j0 in range(D // 16):
            rows_v[r, pl.ds(j0 * 16, 16)] = zer
        return carry

    lax.fori_loop(0, CHUNK, fill_rows, 0)

    def fill_deg(r, carry):
        deg_local[pl.ds(r * 16, 16)] = zer
        return carry

    lax.fori_loop(0, DEG_ROWS // 16, fill_deg, 0)

    pltpu.sync_copy(rows_v, agg_sh.at[pl.ds(base, CHUNK)])
    pltpu.sync_copy(rows_v, agg_sh.at[pl.ds(base + CHUNK, CHUNK)])
    pltpu.sync_copy(rows_v.at[pl.ds(0, 64)], agg_sh.at[pl.ds(base + 2 * CHUNK, 64)])
    plsc.subcore_barrier()

    def body(j, carry):
        pltpu.async_copy(x_hbm.at[src_v.at[j]], rows_v, sem_g).wait()
        pltpu.sync_copy(rows_v, agg_sh.at[dst_v.at[j]], add=True)
        for k in range(CHUNK // 16):
            idx = dst_v[j, pl.ds(k * 16, 16)]
            plsc.addupdate_scatter(deg_local, [idx], one)
        return carry

    lax.fori_loop(0, CHUNKS, body, 0)

    pltpu.sync_copy(deg_local, deg_all.at[pl.ds(s * DEG_ROWS, DEG_ROWS)])
    plsc.subcore_barrier()

    pltpu.sync_copy(agg_sh.at[pl.ds(base, ROWS_PER_TILE)],
                    agg_out.at[c, pl.ds(base, ROWS_PER_TILE)])

    for t in range(NS):
        pltpu.sync_copy(deg_all.at[pl.ds(t * DEG_ROWS + s * SLICE, SLICE)],
                        dsum_v.at[pl.ds(t * SLICE, SLICE)])

    def red(r, carry):
        off = r * 16
        v = dsum_v[pl.ds(off, 16)]
        for t in range(1, NS):
            v = v + dsum_v[pl.ds(t * SLICE + off, 16)]
        dsum_v[pl.ds(off, 16)] = v
        return carry

    lax.fori_loop(0, SLICE // 16, red, 0)
    pltpu.sync_copy(dsum_v.at[pl.ds(0, SLICE)],
                    deg_out.at[pl.ds(c * DEG_ROWS + s * SLICE, SLICE)])


@jax.jit
def _sc_scatter(x, src3, dst4):
    mesh = plsc.VectorSubcoreMesh(core_axis_name="c", subcore_axis_name="s")
    return pl.kernel(
        _sc_body,
        mesh=mesh,
        compiler_params=pltpu.CompilerParams(needs_layout_passes=False),
        out_type=[
            jax.ShapeDtypeStruct((NC, LOCAL_ROWS, D), jnp.float32),
            jax.ShapeDtypeStruct((NC * DEG_ROWS,), jnp.float32),
        ],
        scratch_types=[
            pltpu.VMEM((CHUNKS, CHUNK), jnp.int32),
            pltpu.VMEM((CHUNKS, CHUNK), jnp.int32),
            pltpu.VMEM((CHUNK, D), jnp.float32),
            pltpu.VMEM((DEG_ROWS,), jnp.float32),
            pltpu.VMEM((NS * SLICE,), jnp.float32),
            pltpu.VMEM_SHARED((LOCAL_ROWS, D), jnp.float32),
            pltpu.VMEM_SHARED((NS * DEG_ROWS,), jnp.float32),
            pltpu.SemaphoreType.DMA,
        ],
    )(x, src3, dst4)


def _tc_body(agg_ref, deg_ref, W_ref, b_ref, out_ref):
    h = agg_ref[0] / jnp.maximum(deg_ref[...], 1.0)
    out_ref[...] = (
        jnp.dot(h, W_ref[...], preferred_element_type=jnp.float32) + b_ref[...]
    )


@jax.jit
def _tc_finish(agg2, deg_full, W, b2):
    return pl.pallas_call(
        _tc_body,
        grid=(N // TC_BLK,),
        in_specs=[
            pl.BlockSpec((1, TC_BLK, D), lambda i: (i // 5, i % 5, 0)),
            pl.BlockSpec((TC_BLK, 1), lambda i: (i, 0)),
            pl.BlockSpec((D, D), lambda i: (0, 0)),
            pl.BlockSpec((1, D), lambda i: (0, 0)),
        ],
        out_specs=pl.BlockSpec((TC_BLK, D), lambda i: (i, 0)),
        out_shape=jax.ShapeDtypeStruct((N, D), jnp.float32),
    )(agg2, deg_full, W, b2)


def kernel(x, edge_index, p_map, W, b):
    del p_map
    src = edge_index[0].astype(jnp.int32)
    dst = edge_index[1].astype(jnp.int32)
    pad = E_PAD - E
    src3 = jnp.concatenate([src, jnp.zeros((pad,), jnp.int32)]).reshape(
        NS, CHUNKS, CHUNK)
    dst_p = jnp.concatenate([dst, jnp.full((pad,), -1, jnp.int32)])
    locs = []
    for core in range(NC):
        local = dst_p - core * NSPLIT
        ok = (local >= 0) & (local < NSPLIT)
        locs.append(jnp.where(ok, local, TRASH))
    dst4 = jnp.stack(locs).reshape(NC, NS, CHUNKS, CHUNK)
    agg2, deg2 = _sc_scatter(x, src3, dst4)
    deg_full = jnp.concatenate(
        [deg2[:NSPLIT], deg2[DEG_ROWS:DEG_ROWS + NSPLIT]])[:, None]
    return _tc_finish(agg2, deg_full, W, b.reshape(1, D))


# speedup vs baseline: 16.3692x; 1.2003x over previous
import jax
import jax.numpy as jnp
from jax import lax
from jax.experimental import pallas as pl
from jax.experimental.pallas import tpu as pltpu
from jax.experimental.pallas import tpu_sc as plsc

N = 10000
D = 128
E = 320000
NC = 2
NS = 16
NSPLIT = N // NC
LOCAL_ROWS = 5120
TRASH = NSPLIT
CHUNK = 128
CHUNKS = -(-E // (NS * CHUNK))
E_PAD = NS * CHUNKS * CHUNK
ROWS_PER_TILE = LOCAL_ROWS // NS
SLICE = 320
DEG_ROWS = NS * SLICE
TC_BLK = 1000


def _sc_body(x_hbm, src_hbm, dst_hbm, agg_out, deg_out,
             src_v, dst_v, rows_v, deg_local, dsum_v,
             agg_sh, deg_all, sem_g0, sem_g1):
    c = lax.axis_index("c")
    s = lax.axis_index("s")
    base = s * ROWS_PER_TILE

    pltpu.sync_copy(src_hbm.at[s], src_v)
    pltpu.sync_copy(dst_hbm.at[c, s], dst_v)

    zer = jnp.zeros((16,), jnp.float32)
    one = jnp.ones((16,), jnp.float32)

    def fill_rows(r, carry):
        for j0 in range(D // 16):
            rows_v[0, r, pl.ds(j0 * 16, 16)] = zer
        return carry

    lax.fori_loop(0, CHUNK, fill_rows, 0)

    def fill_deg(r, carry):
        deg_local[pl.ds(r * 16, 16)] = zer
        return carry

    lax.fori_loop(0, DEG_ROWS // 16, fill_deg, 0)

    pltpu.sync_copy(rows_v.at[0], agg_sh.at[pl.ds(base, CHUNK)])
    pltpu.sync_copy(rows_v.at[0], agg_sh.at[pl.ds(base + CHUNK, CHUNK)])
    pltpu.sync_copy(rows_v.at[0, pl.ds(0, 64)],
                    agg_sh.at[pl.ds(base + 2 * CHUNK, 64)])
    plsc.subcore_barrier()

    def hist(j):
        for k in range(CHUNK // 16):
            idx = dst_v[j, pl.ds(k * 16, 16)]
            plsc.addupdate_scatter(deg_local, [idx], one)

    pltpu.async_copy(x_hbm.at[src_v.at[0]], rows_v.at[0], sem_g0)

    def body(i, carry):
        j0 = 2 * i
        j1 = j0 + 1
        j2 = j0 + 2
        pltpu.async_copy(x_hbm.at[src_v.at[j1]], rows_v.at[1], sem_g1)
        pltpu.make_async_copy(x_hbm.at[src_v.at[j0]], rows_v.at[0],
                              sem_g0).wait()
        pltpu.sync_copy(rows_v.at[0], agg_sh.at[dst_v.at[j0]], add=True)
        pltpu.async_copy(x_hbm.at[src_v.at[j2]], rows_v.at[0], sem_g0)
        hist(j0)
        pltpu.make_async_copy(x_hbm.at[src_v.at[j1]], rows_v.at[1],
                              sem_g1).wait()
        pltpu.sync_copy(rows_v.at[1], agg_sh.at[dst_v.at[j1]], add=True)
        hist(j1)
        return carry

    lax.fori_loop(0, (CHUNKS - 1) // 2, body, 0)

    j_last = CHUNKS - 1
    pltpu.make_async_copy(x_hbm.at[src_v.at[j_last]], rows_v.at[0],
                          sem_g0).wait()
    pltpu.sync_copy(rows_v.at[0], agg_sh.at[dst_v.at[j_last]], add=True)
    hist(j_last)

    pltpu.sync_copy(deg_local, deg_all.at[pl.ds(s * DEG_ROWS, DEG_ROWS)])
    plsc.subcore_barrier()

    pltpu.sync_copy(agg_sh.at[pl.ds(base, ROWS_PER_TILE)],
                    agg_out.at[c, pl.ds(base, ROWS_PER_TILE)])

    for t in range(NS):
        pltpu.sync_copy(deg_all.at[pl.ds(t * DEG_ROWS + s * SLICE, SLICE)],
                        dsum_v.at[pl.ds(t * SLICE, SLICE)])

    def red(r, carry):
        off = r * 16
        v = dsum_v[pl.ds(off, 16)]
        for t in range(1, NS):
            v = v + dsum_v[pl.ds(t * SLICE + off, 16)]
        dsum_v[pl.ds(off, 16)] = v
        return carry

    lax.fori_loop(0, SLICE // 16, red, 0)
    pltpu.sync_copy(dsum_v.at[pl.ds(0, SLICE)],
                    deg_out.at[pl.ds(c * DEG_ROWS + s * SLICE, SLICE)])


@jax.jit
def _sc_scatter(x, src3, dst4):
    mesh = plsc.VectorSubcoreMesh(core_axis_name="c", subcore_axis_name="s")
    return pl.kernel(
        _sc_body,
        mesh=mesh,
        compiler_params=pltpu.CompilerParams(needs_layout_passes=False),
        out_type=[
            jax.ShapeDtypeStruct((NC, LOCAL_ROWS, D), jnp.float32),
            jax.ShapeDtypeStruct((NC * DEG_ROWS,), jnp.float32),
        ],
        scratch_types=[
            pltpu.VMEM((CHUNKS, CHUNK), jnp.int32),
            pltpu.VMEM((CHUNKS, CHUNK), jnp.int32),
            pltpu.VMEM((2, CHUNK, D), jnp.float32),
            pltpu.VMEM((DEG_ROWS,), jnp.float32),
            pltpu.VMEM((NS * SLICE,), jnp.float32),
            pltpu.VMEM_SHARED((LOCAL_ROWS, D), jnp.float32),
            pltpu.VMEM_SHARED((NS * DEG_ROWS,), jnp.float32),
            pltpu.SemaphoreType.DMA,
            pltpu.SemaphoreType.DMA,
        ],
    )(x, src3, dst4)


def _tc_body(agg_ref, deg_ref, W_ref, b_ref, out_ref):
    h = agg_ref[0] / jnp.maximum(deg_ref[...], 1.0)
    out_ref[...] = (
        jnp.dot(h, W_ref[...], preferred_element_type=jnp.float32) + b_ref[...]
    )


@jax.jit
def _tc_finish(agg2, deg_full, W, b2):
    return pl.pallas_call(
        _tc_body,
        grid=(N // TC_BLK,),
        in_specs=[
            pl.BlockSpec((1, TC_BLK, D), lambda i: (i // 5, i % 5, 0)),
            pl.BlockSpec((TC_BLK, 1), lambda i: (i, 0)),
            pl.BlockSpec((D, D), lambda i: (0, 0)),
            pl.BlockSpec((1, D), lambda i: (0, 0)),
        ],
        out_specs=pl.BlockSpec((TC_BLK, D), lambda i: (i, 0)),
        out_shape=jax.ShapeDtypeStruct((N, D), jnp.float32),
    )(agg2, deg_full, W, b2)


def kernel(x, edge_index, p_map, W, b):
    del p_map
    src = edge_index[0].astype(jnp.int32)
    dst = edge_index[1].astype(jnp.int32)
    pad = E_PAD - E
    src3 = jnp.concatenate([src, jnp.zeros((pad,), jnp.int32)]).reshape(
        NS, CHUNKS, CHUNK)
    dst_p = jnp.concatenate([dst, jnp.full((pad,), -1, jnp.int32)])
    locs = []
    for core in range(NC):
        local = dst_p - core * NSPLIT
        ok = (local >= 0) & (local < NSPLIT)
        locs.append(jnp.where(ok, local, TRASH))
    dst4 = jnp.stack(locs).reshape(NC, NS, CHUNKS, CHUNK)
    agg2, deg2 = _sc_scatter(x, src3, dst4)
    deg_full = jnp.concatenate(
        [deg2[:NSPLIT], deg2[DEG_ROWS:DEG_ROWS + NSPLIT]])[:, None]
    return _tc_finish(agg2, deg_full, W, b.reshape(1, D))
